# Initial kernel scaffold; baseline (speedup 1.0000x reference)
import jax
import jax.numpy as jnp
from jax.experimental import pallas as pl


def kernel(h, src, dst):
    raise NotImplementedError("write your pallas kernel here")


# speedup vs baseline: 1.2237x; 1.2237x over previous
import jax
import jax.numpy as jnp
from jax import lax
from jax.experimental import pallas as pl
from jax.experimental.pallas import tpu as pltpu
from jax.experimental.pallas import tpu_sc as plsc

N_NODES = 10000
N_EDGES = 320000
D_FEAT = 128
L = 16
NW = 32
E_TILE = N_EDGES // NW
CHUNK = 80
NCHUNK = E_TILE // CHUNK
NGROUP = CHUNK // L


def _rsqrt_nr(x):
    i = plsc.bitcast(x, jnp.int32)
    i = jnp.int32(0x5F3759DF) - lax.shift_right_logical(i, 1)
    y = plsc.bitcast(i, jnp.float32)
    for _ in range(3):
        y = y * (jnp.float32(1.5) - jnp.float32(0.5) * x * y * y)
    return y


def _edge_cosine_body(h_hbm, src_hbm, dst_hbm, out_hbm,
                      idx_s_v, idx_d_v, rows_s_v, rows_d_v, out_v, sem):
    wid = lax.axis_index("s") * 2 + lax.axis_index("c")

    pltpu.sync_copy(src_hbm.at[wid], idx_s_v)
    pltpu.sync_copy(dst_hbm.at[wid], idx_d_v)

    def chunk_body(c, carry):
        cp_s = pltpu.async_copy(h_hbm.at[idx_s_v.at[c]], rows_s_v, sem)
        cp_d = pltpu.async_copy(h_hbm.at[idx_d_v.at[c]], rows_d_v, sem)
        cp_s.wait()
        cp_d.wait()
        for g in range(NGROUP):
            e16 = lax.iota(jnp.int32, L) + (g * L)

            def f_body(f, acc):
                dot, qs, qd = acc
                fv = jnp.broadcast_to(f, (L,))
                s = plsc.load_gather(rows_s_v, [e16, fv])
                d = plsc.load_gather(rows_d_v, [e16, fv])
                return (dot + s * d, qs + s * s, qd + d * d)

            zeros = jnp.zeros((L,), jnp.float32)
            dot, qs, qd = lax.fori_loop(0, D_FEAT, f_body,
                                        (zeros, zeros, zeros))
            denom2 = jnp.maximum(qs * qd, jnp.float32(1e-16))
            res = jnp.maximum(dot * _rsqrt_nr(denom2), jnp.float32(0.0))
            out_v[pl.ds(c * CHUNK + g * L, L)] = res
        return carry

    lax.fori_loop(0, NCHUNK, chunk_body, jnp.int32(0))
    pltpu.sync_copy(out_v, out_hbm.at[pl.ds(wid * E_TILE, E_TILE)])


def kernel(h, src, dst):
    src3 = src.reshape(NW, NCHUNK, CHUNK)
    dst3 = dst.reshape(NW, NCHUNK, CHUNK)
    run = pl.kernel(
        _edge_cosine_body,
        mesh=plsc.VectorSubcoreMesh(core_axis_name="c", subcore_axis_name="s"),
        out_type=jax.ShapeDtypeStruct((N_EDGES,), jnp.float32),
        scratch_types=[
            pltpu.VMEM((NCHUNK, CHUNK), jnp.int32),
            pltpu.VMEM((NCHUNK, CHUNK), jnp.int32),
            pltpu.VMEM((CHUNK, D_FEAT), jnp.float32),
            pltpu.VMEM((CHUNK, D_FEAT), jnp.float32),
            pltpu.VMEM((E_TILE,), jnp.float32),
            pltpu.SemaphoreType.DMA,
        ],
        compiler_params=pltpu.CompilerParams(needs_layout_passes=False),
    )
    return run(h, src3, dst3)


# speedup vs baseline: 1.4022x; 1.1459x over previous
import jax
import jax.numpy as jnp
from jax import lax
from jax.experimental import pallas as pl
from jax.experimental.pallas import tpu as pltpu
from jax.experimental.pallas import tpu_sc as plsc

N_NODES = 10000
N_EDGES = 320000
D_FEAT = 128
L = 16
NW = 32
E_TILE = N_EDGES // NW
CHUNK = 80
NCHUNK = E_TILE // CHUNK
NGROUP = CHUNK // L


def _rsqrt_nr(x):
    i = plsc.bitcast(x, jnp.int32)
    i = jnp.int32(0x5F3759DF) - lax.shift_right_logical(i, 1)
    y = plsc.bitcast(i, jnp.float32)
    for _ in range(3):
        y = y * (jnp.float32(1.5) - jnp.float32(0.5) * x * y * y)
    return y


def _edge_cosine_body(h_hbm, src_hbm, dst_hbm, out_hbm,
                      idx_s_v, idx_d_v, rows_s_v, rows_d_v, out_v,
                      sem0, sem1):
    wid = lax.axis_index("s") * 2 + lax.axis_index("c")

    pltpu.sync_copy(src_hbm.at[wid], idx_s_v)
    pltpu.sync_copy(dst_hbm.at[wid], idx_d_v)

    sems = (sem0, sem1)

    def start(c, slot):
        sem = sems[slot]
        pltpu.async_copy(h_hbm.at[idx_s_v.at[c]], rows_s_v.at[slot], sem)
        pltpu.async_copy(h_hbm.at[idx_d_v.at[c]], rows_d_v.at[slot], sem)

    def wait(slot):
        sem = sems[slot]
        pltpu.make_async_copy(
            h_hbm.at[idx_s_v.at[0]], rows_s_v.at[slot], sem).wait()
        pltpu.make_async_copy(
            h_hbm.at[idx_d_v.at[0]], rows_d_v.at[slot], sem).wait()

    def compute(c, slot):
        rs = rows_s_v.at[slot]
        rd = rows_d_v.at[slot]
        for g in range(NGROUP):
            e16 = lax.iota(jnp.int32, L) + (g * L)

            def f_body(f, acc):
                del f
                dot, qs, qd, fv = acc
                s = plsc.load_gather(rs, [e16, fv])
                d = plsc.load_gather(rd, [e16, fv])
                return (dot + s * d, qs + s * s, qd + d * d, fv + 1)

            zeros = jnp.zeros((L,), jnp.float32)
            dot, qs, qd, _ = lax.fori_loop(
                0, D_FEAT, f_body,
                (zeros, zeros, zeros, jnp.zeros((L,), jnp.int32)),
                unroll=8)
            denom2 = jnp.maximum(qs * qd, jnp.float32(1e-16))
            res = jnp.maximum(dot * _rsqrt_nr(denom2), jnp.float32(0.0))
            out_v[pl.ds(c * CHUNK + g * L, L)] = res

    start(0, 0)

    def pair_body(i, carry):
        c0 = 2 * i
        start(c0 + 1, 1)
        wait(0)
        compute(c0, 0)
        start(c0 + 2, 0)
        wait(1)
        compute(c0 + 1, 1)
        return carry

    lax.fori_loop(0, (NCHUNK - 1) // 2, pair_body, jnp.int32(0))
    wait(0)
    compute(NCHUNK - 1, 0)

    pltpu.sync_copy(out_v, out_hbm.at[pl.ds(wid * E_TILE, E_TILE)])


def kernel(h, src, dst):
    src3 = src.reshape(NW, NCHUNK, CHUNK)
    dst3 = dst.reshape(NW, NCHUNK, CHUNK)
    run = pl.kernel(
        _edge_cosine_body,
        mesh=plsc.VectorSubcoreMesh(core_axis_name="c", subcore_axis_name="s"),
        out_type=jax.ShapeDtypeStruct((N_EDGES,), jnp.float32),
        scratch_types=[
            pltpu.VMEM((NCHUNK, CHUNK), jnp.int32),
            pltpu.VMEM((NCHUNK, CHUNK), jnp.int32),
            pltpu.VMEM((2, CHUNK, D_FEAT), jnp.float32),
            pltpu.VMEM((2, CHUNK, D_FEAT), jnp.float32),
            pltpu.VMEM((E_TILE,), jnp.float32),
            pltpu.SemaphoreType.DMA,
            pltpu.SemaphoreType.DMA,
        ],
        compiler_params=pltpu.CompilerParams(needs_layout_passes=False),
    )
    return run(h, src3, dst3)
